# Initial kernel scaffold; baseline (speedup 1.0000x reference)
import jax
import jax.numpy as jnp
from jax.experimental import pallas as pl


def kernel(V, E, edge_index, params):
    raise NotImplementedError("write your pallas kernel here")


# speedup vs baseline: 2.2418x; 2.2418x over previous
import functools

import jax
import jax.numpy as jnp
from jax import lax
from jax.experimental import pallas as pl
from jax.experimental.pallas import tpu as pltpu
from jax.experimental.pallas import tpu_sc as plsc

N_NODES = 10000
NPAD = 10240
N_EDGES = 320000
H = 128
ROW_TILE = 512
IDX_ROWS = N_EDGES // 128
SLAB = 4
N_SLABS = IDX_ROWS // SLAB
SC_SLAB = 2
N_SC_SLABS = IDX_ROWS // SC_SLAB
NW = 32

_F32 = jnp.float32


def _dot(x, w):
    return jnp.dot(x, w, preferred_element_type=_F32)


def _leaky(x):
    return jnp.where(x >= 0, x, 0.01 * x)


def _layernorm(x, g, b):
    mu = jnp.mean(x, axis=-1, keepdims=True)
    xc = x - mu
    var = jnp.mean(xc * xc, axis=-1, keepdims=True)
    return xc * lax.rsqrt(var + 1e-5) * g + b


def _tail3(x, w1, b1, w2, b2, w3, b3, g, bln):
    x = _leaky(x)
    x = _dot(x, w1[...]) + b1[...]
    x = _leaky(x)
    x = _dot(x, w2[...]) + b2[...]
    x = _leaky(x)
    x = _dot(x, w3[...]) + b3[...]
    return _layernorm(x, g[...], bln[...])


def _enc_body(x_ref, w0, b0, w1, b1, w2, b2, w3, b3, g, bln, o_ref):
    x = _dot(x_ref[...], w0[...]) + b0[...]
    o_ref[...] = _tail3(x, w1, b1, w2, b2, w3, b3, g, bln)


def _msg_body(vd, vs, le, w0, b0, w1, b1, w2, b2, w3, b3, g, bln, msg_o, le_o):
    x = (_dot(vd[...], w0[0:H, :]) + _dot(vs[...], w0[H:2 * H, :])
         + _dot(le[...], w0[2 * H:3 * H, :]) + b0[...])
    m = _tail3(x, w1, b1, w2, b2, w3, b3, g, bln)
    msg_o[...] = m
    le_o[...] = le[...] + m


def _upd_body(a0, a1, v, w0, b0, w1, b1, w2, b2, w3, b3, g, bln, o_ref):
    agg = a0[...] + a1[...]
    x = _dot(agg, w0[0:H, :]) + _dot(v[...], w0[H:2 * H, :]) + b0[...]
    o_ref[...] = v[...] + _tail3(x, w1, b1, w2, b2, w3, b3, g, bln)


def _dec_body(x_ref, w0, b0, w1, b1, w2, b2, w3, b3, o_ref):
    x = _leaky(_dot(x_ref[...], w0[...]) + b0[...])
    x = _leaky(_dot(x, w1[...]) + b1[...])
    x = _leaky(_dot(x, w2[...]) + b2[...])
    o_ref[...] = _dot(x, w3[...]) + b3[...]


def _mlp_call(body, parts, weights, out_shapes):
    n_rows = out_shapes[0][0]
    grid = (n_rows // ROW_TILE,)

    def _row_spec(ncols, off):
        return pl.BlockSpec((ROW_TILE, ncols), lambda i, o=off: (i + o, 0))

    in_specs = ([_row_spec(p.shape[1], off) for p, off in parts]
                + [pl.BlockSpec(w.shape, lambda i: (0, 0)) for w in weights])
    out_specs = [_row_spec(s[1], 0) for s in out_shapes]
    out_shape = [jax.ShapeDtypeStruct(s, _F32) for s in out_shapes]
    single = len(out_shapes) == 1
    res = pl.pallas_call(
        body,
        grid=grid,
        in_specs=in_specs,
        out_specs=out_specs[0] if single else out_specs,
        out_shape=out_shape[0] if single else out_shape,
        compiler_params=pltpu.CompilerParams(
            dimension_semantics=("parallel",)),
    )(*[p for p, _ in parts], *weights)
    return res


def _prep_mlp(p):
    ws = []
    for l in p["linears"]:
        ws.append(jnp.asarray(l["W"].T, _F32))
        ws.append(l["b"].reshape(1, -1).astype(_F32))
    if "ln" in p:
        ws.append(p["ln"]["g"].reshape(1, -1).astype(_F32))
        ws.append(p["ln"]["b"].reshape(1, -1).astype(_F32))
    return ws


def _sc_mesh():
    return plsc.VectorSubcoreMesh(core_axis_name="c", subcore_axis_name="s")


def _sc_gather_pair(table, dst2d, src2d):
    @functools.partial(
        pl.kernel,
        mesh=_sc_mesh(),
        out_type=[jax.ShapeDtypeStruct((N_EDGES, H), _F32),
                  jax.ShapeDtypeStruct((N_EDGES, H), _F32)],
        scratch_types=[pltpu.VMEM((SLAB, 128), jnp.int32),
                       pltpu.VMEM((SLAB * 128, H), _F32),
                       pltpu.SemaphoreType.DMA],
    )
    def gk(table_hbm, d_hbm, s_hbm, od_hbm, os_hbm, idx_v, rows_v, sem):
        wid = lax.axis_index("s") * 2 + lax.axis_index("c")

        @pl.loop(wid, N_SLABS, step=NW)
        def _(k):
            for ih, oh in ((d_hbm, od_hbm), (s_hbm, os_hbm)):
                pltpu.sync_copy(ih.at[pl.ds(k * SLAB, SLAB)], idx_v)
                cps = [pltpu.async_copy(table_hbm.at[idx_v.at[j]],
                                        rows_v.at[pl.ds(j * 128, 128)], sem)
                       for j in range(SLAB)]
                for cp in cps:
                    cp.wait()
                pltpu.sync_copy(rows_v, oh.at[pl.ds(k * SLAB * 128, SLAB * 128)])

    return gk(table, dst2d, src2d)


def _sc_scatter_add(msg, dst2d, zeros):
    @functools.partial(
        pl.kernel,
        mesh=_sc_mesh(),
        out_type=jax.ShapeDtypeStruct((2 * NPAD, H), _F32),
        scratch_types=[pltpu.VMEM((SC_SLAB, 128), jnp.int32),
                       pltpu.VMEM((SC_SLAB * 128, H), _F32),
                       pltpu.VMEM_SHARED((NPAD, H), _F32)],
    )
    def sk(msg_hbm, d_hbm, z_hbm, out_hbm, idx_v, rows_v, acc):
        c = lax.axis_index("c")
        s = lax.axis_index("s")
        wid = s * 2 + c
        rows_per = NPAD // 16
        pltpu.sync_copy(z_hbm.at[pl.ds(s * rows_per, rows_per)],
                        acc.at[pl.ds(s * rows_per, rows_per)])
        plsc.subcore_barrier()

        @pl.loop(wid, N_SC_SLABS, step=NW)
        def _(k):
            pltpu.sync_copy(d_hbm.at[pl.ds(k * SC_SLAB, SC_SLAB)], idx_v)
            pltpu.sync_copy(msg_hbm.at[pl.ds(k * SC_SLAB * 128, SC_SLAB * 128)],
                            rows_v)
            for j in range(SC_SLAB):
                pltpu.sync_copy(rows_v.at[pl.ds(j * 128, 128)],
                                acc.at[idx_v.at[j]], add=True)

        plsc.subcore_barrier()
        pltpu.sync_copy(acc.at[pl.ds(s * rows_per, rows_per)],
                        out_hbm.at[pl.ds(c * NPAD + s * rows_per, rows_per)])

    return sk(msg, dst2d, zeros)


def kernel(V, E, edge_index, params):
    src2d = edge_index[0].reshape(IDX_ROWS, 128)
    dst2d = edge_index[1].reshape(IDX_ROWS, 128)
    Vp = jnp.pad(V, ((0, NPAD - N_NODES), (0, 0)))
    zeros = jnp.zeros((NPAD, H), _F32)

    lV = _mlp_call(_enc_body, [(Vp, 0)], _prep_mlp(params["enc_V"]),
                   [(NPAD, H)])
    lE = _mlp_call(_enc_body, [(E, 0)], _prep_mlp(params["enc_E"]),
                   [(N_EDGES, H)])

    for p in params["procs"]:
        Vd, Vs = _sc_gather_pair(lV, dst2d, src2d)
        msg, lE = _mlp_call(_msg_body, [(Vd, 0), (Vs, 0), (lE, 0)],
                            _prep_mlp(p["msg"]),
                            [(N_EDGES, H), (N_EDGES, H)])
        aggs = _sc_scatter_add(msg, dst2d, zeros)
        lV = _mlp_call(_upd_body,
                       [(aggs, 0), (aggs, NPAD // ROW_TILE), (lV, 0)],
                       _prep_mlp(p["upd"]), [(NPAD, H)])

    dec = _prep_mlp(params["dec"])
    dec[6] = jnp.pad(dec[6], ((0, 0), (0, 5)))
    dec[7] = jnp.pad(dec[7], ((0, 0), (0, 5)))
    out = _mlp_call(_dec_body, [(lV, 0)], dec, [(NPAD, 8)])
    return out[:N_NODES, :3]


# speedup vs baseline: 2.2543x; 1.0055x over previous
import functools

import jax
import jax.numpy as jnp
from jax import lax
from jax.experimental import pallas as pl
from jax.experimental.pallas import tpu as pltpu
from jax.experimental.pallas import tpu_sc as plsc

N_NODES = 10000
NPAD = 10240
N_EDGES = 320000
H = 128
ROW_TILE = 512
IDX_ROWS = N_EDGES // 128
GSLAB = 2
N_GSLABS = IDX_ROWS // GSLAB
SC_SLAB = 2
N_SC_SLABS = IDX_ROWS // SC_SLAB
NW = 32

_F32 = jnp.float32


def _dot(x, w):
    return jnp.dot(x.astype(jnp.bfloat16), w, preferred_element_type=_F32)


def _leaky(x):
    return jnp.where(x >= 0, x, 0.01 * x)


def _layernorm(x, g, b):
    mu = jnp.mean(x, axis=-1, keepdims=True)
    xc = x - mu
    var = jnp.mean(xc * xc, axis=-1, keepdims=True)
    return xc * lax.rsqrt(var + 1e-5) * g + b


def _tail3(x, w1, b1, w2, b2, w3, b3, g, bln):
    x = _leaky(x)
    x = _dot(x, w1[...]) + b1[...]
    x = _leaky(x)
    x = _dot(x, w2[...]) + b2[...]
    x = _leaky(x)
    x = _dot(x, w3[...]) + b3[...]
    return _layernorm(x, g[...], bln[...])


def _enc_body(x_ref, w0, b0, w1, b1, w2, b2, w3, b3, g, bln, o_ref):
    x = _dot(x_ref[...], w0[...]) + b0[...]
    o_ref[...] = _tail3(x, w1, b1, w2, b2, w3, b3, g, bln)


def _enc_dual_body(x_ref, w0, b0, w1, b1, w2, b2, w3, b3, g, bln, o_ref, ob_ref):
    x = _dot(x_ref[...], w0[...]) + b0[...]
    y = _tail3(x, w1, b1, w2, b2, w3, b3, g, bln)
    o_ref[...] = y
    ob_ref[...] = y.astype(jnp.bfloat16)


def _msg_body(vd, vs, le, w0, b0, w1, b1, w2, b2, w3, b3, g, bln, msg_o, le_o):
    x = (_dot(vd[...], w0[0:H, :]) + _dot(vs[...], w0[H:2 * H, :])
         + _dot(le[...], w0[2 * H:3 * H, :]) + b0[...])
    m = _tail3(x, w1, b1, w2, b2, w3, b3, g, bln)
    msg_o[...] = m
    le_o[...] = le[...] + m


def _upd_body(a0, a1, v, w0, b0, w1, b1, w2, b2, w3, b3, g, bln, o_ref, ob_ref):
    agg = a0[...] + a1[...]
    x = _dot(agg, w0[0:H, :]) + _dot(v[...], w0[H:2 * H, :]) + b0[...]
    y = v[...] + _tail3(x, w1, b1, w2, b2, w3, b3, g, bln)
    o_ref[...] = y
    ob_ref[...] = y.astype(jnp.bfloat16)


def _dec_body(x_ref, w0, b0, w1, b1, w2, b2, w3, b3, o_ref):
    x = _leaky(_dot(x_ref[...], w0[...]) + b0[...])
    x = _leaky(_dot(x, w1[...]) + b1[...])
    x = _leaky(_dot(x, w2[...]) + b2[...])
    o_ref[...] = _dot(x, w3[...]) + b3[...]


def _mlp_call(body, parts, weights, outs):
    n_rows = outs[0].shape[0]
    grid = (n_rows // ROW_TILE,)

    def _row_spec(ncols, off):
        return pl.BlockSpec((ROW_TILE, ncols), lambda i, o=off: (i + o, 0))

    in_specs = ([_row_spec(p.shape[1], off) for p, off in parts]
                + [pl.BlockSpec(w.shape, lambda i: (0, 0)) for w in weights])
    out_specs = [_row_spec(o.shape[1], 0) for o in outs]
    single = len(outs) == 1
    res = pl.pallas_call(
        body,
        grid=grid,
        in_specs=in_specs,
        out_specs=out_specs[0] if single else out_specs,
        out_shape=outs[0] if single else outs,
        compiler_params=pltpu.CompilerParams(
            dimension_semantics=("parallel",)),
    )(*[p for p, _ in parts], *weights)
    return res


def _prep_mlp(p):
    ws = []
    for l in p["linears"]:
        ws.append(l["W"].T.astype(jnp.bfloat16))
        ws.append(l["b"].reshape(1, -1).astype(_F32))
    if "ln" in p:
        ws.append(p["ln"]["g"].reshape(1, -1).astype(_F32))
        ws.append(p["ln"]["b"].reshape(1, -1).astype(_F32))
    return ws


def _sc_mesh():
    return plsc.VectorSubcoreMesh(core_axis_name="c", subcore_axis_name="s")


def _sc_gather_pair(table, dst2d, src2d):
    @functools.partial(
        pl.kernel,
        mesh=_sc_mesh(),
        out_type=[jax.ShapeDtypeStruct((N_EDGES, H), _F32),
                  jax.ShapeDtypeStruct((N_EDGES, H), _F32)],
        scratch_types=[pltpu.VMEM((GSLAB, 128), jnp.int32),
                       pltpu.VMEM((GSLAB, 128), jnp.int32),
                       pltpu.VMEM((GSLAB * 128, H), _F32),
                       pltpu.VMEM((GSLAB * 128, H), _F32),
                       pltpu.SemaphoreType.DMA,
                       pltpu.SemaphoreType.DMA,
                       pltpu.SemaphoreType.DMA,
                       pltpu.SemaphoreType.DMA],
    )
    def gk(table_hbm, d_hbm, s_hbm, od_hbm, os_hbm,
           idx_d, idx_s, rows_d, rows_s, sem_d, sem_s, sem_od, sem_os):
        wid = lax.axis_index("s") * 2 + lax.axis_index("c")

        @pl.loop(wid, N_GSLABS, step=NW)
        def _(k):
            pltpu.sync_copy(d_hbm.at[pl.ds(k * GSLAB, GSLAB)], idx_d)
            gd = [pltpu.async_copy(table_hbm.at[idx_d.at[j]],
                                   rows_d.at[pl.ds(j * 128, 128)], sem_d)
                  for j in range(GSLAB)]
            pltpu.sync_copy(s_hbm.at[pl.ds(k * GSLAB, GSLAB)], idx_s)
            gs = [pltpu.async_copy(table_hbm.at[idx_s.at[j]],
                                   rows_s.at[pl.ds(j * 128, 128)], sem_s)
                  for j in range(GSLAB)]
            for cp in gd:
                cp.wait()
            od = pltpu.async_copy(rows_d,
                                  od_hbm.at[pl.ds(k * GSLAB * 128, GSLAB * 128)],
                                  sem_od)
            for cp in gs:
                cp.wait()
            os_ = pltpu.async_copy(rows_s,
                                   os_hbm.at[pl.ds(k * GSLAB * 128, GSLAB * 128)],
                                   sem_os)
            od.wait()
            os_.wait()

    return gk(table, dst2d, src2d)


def _sc_scatter_add(msg, dst2d, zeros):
    @functools.partial(
        pl.kernel,
        mesh=_sc_mesh(),
        out_type=jax.ShapeDtypeStruct((2 * NPAD, H), _F32),
        scratch_types=[pltpu.VMEM((SC_SLAB, 128), jnp.int32),
                       pltpu.VMEM((SC_SLAB * 128, H), _F32),
                       pltpu.VMEM_SHARED((NPAD, H), _F32)],
    )
    def sk(msg_hbm, d_hbm, z_hbm, out_hbm, idx_v, rows_v, acc):
        c = lax.axis_index("c")
        s = lax.axis_index("s")
        wid = s * 2 + c
        rows_per = NPAD // 16
        pltpu.sync_copy(z_hbm.at[pl.ds(s * rows_per, rows_per)],
                        acc.at[pl.ds(s * rows_per, rows_per)])
        plsc.subcore_barrier()

        @pl.loop(wid, N_SC_SLABS, step=NW)
        def _(k):
            pltpu.sync_copy(d_hbm.at[pl.ds(k * SC_SLAB, SC_SLAB)], idx_v)
            pltpu.sync_copy(msg_hbm.at[pl.ds(k * SC_SLAB * 128, SC_SLAB * 128)],
                            rows_v)
            for j in range(SC_SLAB):
                pltpu.sync_copy(rows_v.at[pl.ds(j * 128, 128)],
                                acc.at[idx_v.at[j]], add=True)

        plsc.subcore_barrier()
        pltpu.sync_copy(acc.at[pl.ds(s * rows_per, rows_per)],
                        out_hbm.at[pl.ds(c * NPAD + s * rows_per, rows_per)])

    return sk(msg, dst2d, zeros)


def kernel(V, E, edge_index, params):
    src2d = edge_index[0].reshape(IDX_ROWS, 128)
    dst2d = edge_index[1].reshape(IDX_ROWS, 128)
    Vp = jnp.pad(V, ((0, NPAD - N_NODES), (0, 0)))
    zeros = jnp.zeros((NPAD, H), _F32)

    _sd = jax.ShapeDtypeStruct
    bf16 = jnp.bfloat16
    lV, lVb = _mlp_call(_enc_dual_body, [(Vp, 0)], _prep_mlp(params["enc_V"]),
                        [_sd((NPAD, H), _F32), _sd((NPAD, H), bf16)])
    lE = _mlp_call(_enc_body, [(E, 0)], _prep_mlp(params["enc_E"]),
                   [_sd((N_EDGES, H), _F32)])

    for p in params["procs"]:
        Vd, Vs = _sc_gather_pair(lV, dst2d, src2d)
        msg, lE = _mlp_call(_msg_body, [(Vd, 0), (Vs, 0), (lE, 0)],
                            _prep_mlp(p["msg"]),
                            [_sd((N_EDGES, H), _F32), _sd((N_EDGES, H), _F32)])
        aggs = _sc_scatter_add(msg, dst2d, zeros)
        lV, lVb = _mlp_call(_upd_body,
                            [(aggs, 0), (aggs, NPAD // ROW_TILE), (lV, 0)],
                            _prep_mlp(p["upd"]),
                            [_sd((NPAD, H), _F32), _sd((NPAD, H), bf16)])

    dec = _prep_mlp(params["dec"])
    dec[6] = jnp.pad(dec[6], ((0, 0), (0, 5)))
    dec[7] = jnp.pad(dec[7], ((0, 0), (0, 5)))
    out = _mlp_call(_dec_body, [(lV, 0)], dec, [_sd((NPAD, 8), _F32)])
    return out[:N_NODES, :3]


# speedup vs baseline: 2.3208x; 1.0295x over previous
import functools

import jax
import jax.numpy as jnp
from jax import lax
from jax.experimental import pallas as pl
from jax.experimental.pallas import tpu as pltpu
from jax.experimental.pallas import tpu_sc as plsc

N_NODES = 10000
NPAD = 10240
N_EDGES = 320000
H = 128
ROW_TILE = 512
IDX_ROWS = N_EDGES // 128
GSLAB = 2
N_GSLABS = IDX_ROWS // GSLAB
HP = H // 2
SC_SLAB = 2
N_SC_SLABS = IDX_ROWS // SC_SLAB
NW = 32

_F32 = jnp.float32


def _dot(x, w):
    return jnp.dot(x.astype(jnp.bfloat16), w, preferred_element_type=_F32)


def _leaky(x):
    return jnp.where(x >= 0, x, 0.01 * x)


def _layernorm(x, g, b):
    mu = jnp.mean(x, axis=-1, keepdims=True)
    xc = x - mu
    var = jnp.mean(xc * xc, axis=-1, keepdims=True)
    return xc * lax.rsqrt(var + 1e-5) * g + b


def _tail3(x, w1, b1, w2, b2, w3, b3, g, bln):
    x = _leaky(x)
    x = _dot(x, w1[...]) + b1[...]
    x = _leaky(x)
    x = _dot(x, w2[...]) + b2[...]
    x = _leaky(x)
    x = _dot(x, w3[...]) + b3[...]
    return _layernorm(x, g[...], bln[...])


def _enc_body(x_ref, w0, b0, w1, b1, w2, b2, w3, b3, g, bln, o_ref):
    x = _dot(x_ref[...], w0[...]) + b0[...]
    o_ref[...] = _tail3(x, w1, b1, w2, b2, w3, b3, g, bln)


def _unpack2(xi):
    lo = lax.bitcast_convert_type(xi << 16, _F32)
    hi = lax.bitcast_convert_type(xi & jnp.int32(-65536), _F32)
    return lo, hi


def _enc_bf_body(x_ref, w0, b0, w1, b1, w2, b2, w3, b3, g, bln, o_ref):
    x = _dot(x_ref[...], w0[...]) + b0[...]
    o_ref[...] = _tail3(x, w1, b1, w2, b2, w3, b3, g, bln).astype(jnp.bfloat16)


def _enc_dual_body(x_ref, w0, b0, w1, b1, w2, b2, w3, b3, g, bln, o_ref, ob_ref):
    x = _dot(x_ref[...], w0[...]) + b0[...]
    y = _tail3(x, w1, b1, w2, b2, w3, b3, g, bln)
    o_ref[...] = y
    ob_ref[...] = y.astype(jnp.bfloat16)


def _msg_body(vd, vs, le, wde, wdo, wse, wso, wle, b0,
              w1, b1, w2, b2, w3, b3, g, bln, msg_o, le_o):
    de, do = _unpack2(vd[...])
    se, so = _unpack2(vs[...])
    x = (_dot(de, wde[...]) + _dot(do, wdo[...])
         + _dot(se, wse[...]) + _dot(so, wso[...])
         + _dot(le[...], wle[...]) + b0[...])
    m = _tail3(x, w1, b1, w2, b2, w3, b3, g, bln)
    msg_o[...] = m
    le_o[...] = (le[...].astype(_F32) + m).astype(jnp.bfloat16)


def _msg_f32_body(vd, vs, le, w0, b0, w1, b1, w2, b2, w3, b3, g, bln,
                  msg_o, le_o):
    x = (_dot(vd[...], w0[0:H, :]) + _dot(vs[...], w0[H:2 * H, :])
         + _dot(le[...], w0[2 * H:3 * H, :]) + b0[...])
    m = _tail3(x, w1, b1, w2, b2, w3, b3, g, bln)
    msg_o[...] = m
    le_o[...] = (le[...].astype(_F32) + m).astype(jnp.bfloat16)


def _upd_body(a0, a1, v, w0, b0, w1, b1, w2, b2, w3, b3, g, bln, o_ref, ob_ref):
    agg = a0[...] + a1[...]
    x = _dot(agg, w0[0:H, :]) + _dot(v[...], w0[H:2 * H, :]) + b0[...]
    y = v[...] + _tail3(x, w1, b1, w2, b2, w3, b3, g, bln)
    o_ref[...] = y
    ob_ref[...] = y.astype(jnp.bfloat16)


def _dec_body(x_ref, w0, b0, w1, b1, w2, b2, w3, b3, o_ref):
    x = _leaky(_dot(x_ref[...], w0[...]) + b0[...])
    x = _leaky(_dot(x, w1[...]) + b1[...])
    x = _leaky(_dot(x, w2[...]) + b2[...])
    o_ref[...] = _dot(x, w3[...]) + b3[...]


def _mlp_call(body, parts, weights, outs):
    n_rows = outs[0].shape[0]
    grid = (n_rows // ROW_TILE,)

    def _row_spec(ncols, off):
        return pl.BlockSpec((ROW_TILE, ncols), lambda i, o=off: (i + o, 0))

    in_specs = ([_row_spec(p.shape[1], off) for p, off in parts]
                + [pl.BlockSpec(w.shape, lambda i: (0, 0)) for w in weights])
    out_specs = [_row_spec(o.shape[1], 0) for o in outs]
    single = len(outs) == 1
    res = pl.pallas_call(
        body,
        grid=grid,
        in_specs=in_specs,
        out_specs=out_specs[0] if single else out_specs,
        out_shape=outs[0] if single else outs,
        compiler_params=pltpu.CompilerParams(
            dimension_semantics=("parallel",)),
    )(*[p for p, _ in parts], *weights)
    return res


def _prep_mlp(p):
    ws = []
    for l in p["linears"]:
        ws.append(l["W"].T.astype(jnp.bfloat16))
        ws.append(l["b"].reshape(1, -1).astype(_F32))
    if "ln" in p:
        ws.append(p["ln"]["g"].reshape(1, -1).astype(_F32))
        ws.append(p["ln"]["b"].reshape(1, -1).astype(_F32))
    return ws


def _prep_msg(p):
    ws = _prep_mlp(p)
    w0 = ws[0]
    wd, wsr, wle = w0[0:H], w0[H:2 * H], w0[2 * H:3 * H]
    return [wd[0::2], wd[1::2], wsr[0::2], wsr[1::2], wle] + ws[1:]


def _sc_mesh():
    return plsc.VectorSubcoreMesh(core_axis_name="c", subcore_axis_name="s")


def _sc_gather_pair(table, dst1, src1):
    SLAB_E = GSLAB * 128

    @functools.partial(
        pl.kernel,
        mesh=_sc_mesh(),
        out_type=[jax.ShapeDtypeStruct((N_EDGES, H), _F32),
                  jax.ShapeDtypeStruct((N_EDGES, H), _F32)],
        scratch_types=[pltpu.VMEM((SLAB_E,), jnp.int32),
                       pltpu.VMEM((SLAB_E,), jnp.int32),
                       pltpu.VMEM((SLAB_E, H), _F32),
                       pltpu.VMEM((SLAB_E, H), _F32),
                       pltpu.SemaphoreType.DMA,
                       pltpu.SemaphoreType.DMA,
                       pltpu.SemaphoreType.DMA,
                       pltpu.SemaphoreType.DMA],
    )
    def gk(table_hbm, d_hbm, s_hbm, od_hbm, os_hbm,
           idx_d, idx_s, rows_d, rows_s, sem_d, sem_s, sem_od, sem_os):
        wid = lax.axis_index("s") * 2 + lax.axis_index("c")

        @pl.loop(wid, N_GSLABS, step=NW)
        def _(k):
            base = k * SLAB_E
            pltpu.sync_copy(d_hbm.at[pl.ds(base, SLAB_E)], idx_d)
            gd = [pltpu.async_copy(table_hbm.at[idx_d.at[pl.ds(j * 128, 128)]],
                                   rows_d.at[pl.ds(j * 128, 128)], sem_d)
                  for j in range(GSLAB)]
            pltpu.sync_copy(s_hbm.at[pl.ds(base, SLAB_E)], idx_s)
            gs = [pltpu.async_copy(table_hbm.at[idx_s.at[pl.ds(j * 128, 128)]],
                                   rows_s.at[pl.ds(j * 128, 128)], sem_s)
                  for j in range(GSLAB)]
            for cp in gd:
                cp.wait()
            od = pltpu.async_copy(rows_d, od_hbm.at[pl.ds(base, SLAB_E)],
                                  sem_od)
            for cp in gs:
                cp.wait()
            os_ = pltpu.async_copy(rows_s, os_hbm.at[pl.ds(base, SLAB_E)],
                                   sem_os)
            od.wait()
            os_.wait()

    return gk(table, dst1, src1)


def _sc_scatter_add(msg, dst2d, zeros):
    @functools.partial(
        pl.kernel,
        mesh=_sc_mesh(),
        out_type=jax.ShapeDtypeStruct((2 * NPAD, H), _F32),
        scratch_types=[pltpu.VMEM((SC_SLAB, 128), jnp.int32),
                       pltpu.VMEM((SC_SLAB * 128, H), _F32),
                       pltpu.VMEM_SHARED((NPAD, H), _F32)],
    )
    def sk(msg_hbm, d_hbm, z_hbm, out_hbm, idx_v, rows_v, acc):
        c = lax.axis_index("c")
        s = lax.axis_index("s")
        wid = s * 2 + c
        rows_per = NPAD // 16
        pltpu.sync_copy(z_hbm.at[pl.ds(s * rows_per, rows_per)],
                        acc.at[pl.ds(s * rows_per, rows_per)])
        plsc.subcore_barrier()

        @pl.loop(wid, N_SC_SLABS, step=NW)
        def _(k):
            pltpu.sync_copy(d_hbm.at[pl.ds(k * SC_SLAB, SC_SLAB)], idx_v)
            pltpu.sync_copy(msg_hbm.at[pl.ds(k * SC_SLAB * 128, SC_SLAB * 128)],
                            rows_v)
            for j in range(SC_SLAB):
                pltpu.sync_copy(rows_v.at[pl.ds(j * 128, 128)],
                                acc.at[idx_v.at[j]], add=True)

        plsc.subcore_barrier()
        pltpu.sync_copy(acc.at[pl.ds(s * rows_per, rows_per)],
                        out_hbm.at[pl.ds(c * NPAD + s * rows_per, rows_per)])

    return sk(msg, dst2d, zeros)


def kernel(V, E, edge_index, params):
    src1 = edge_index[0]
    dst1 = edge_index[1]
    dst2d = edge_index[1].reshape(IDX_ROWS, 128)
    Vp = jnp.pad(V, ((0, NPAD - N_NODES), (0, 0)))
    zeros = jnp.zeros((NPAD, H), _F32)

    _sd = jax.ShapeDtypeStruct
    bf16 = jnp.bfloat16
    lV, lVb = _mlp_call(_enc_dual_body, [(Vp, 0)], _prep_mlp(params["enc_V"]),
                        [_sd((NPAD, H), _F32), _sd((NPAD, H), bf16)])
    lE = _mlp_call(_enc_bf_body, [(E, 0)], _prep_mlp(params["enc_E"]),
                   [_sd((N_EDGES, H), bf16)])

    for p in params["procs"]:
        Vd, Vs = _sc_gather_pair(lV, dst1, src1)
        msg, lE = _mlp_call(_msg_f32_body, [(Vd, 0), (Vs, 0), (lE, 0)],
                            _prep_mlp(p["msg"]),
                            [_sd((N_EDGES, H), _F32), _sd((N_EDGES, H), bf16)])
        aggs = _sc_scatter_add(msg, dst2d, zeros)
        lV, lVb = _mlp_call(_upd_body,
                            [(aggs, 0), (aggs, NPAD // ROW_TILE), (lV, 0)],
                            _prep_mlp(p["upd"]),
                            [_sd((NPAD, H), _F32), _sd((NPAD, H), bf16)])

    dec = _prep_mlp(params["dec"])
    dec[6] = jnp.pad(dec[6], ((0, 0), (0, 5)))
    dec[7] = jnp.pad(dec[7], ((0, 0), (0, 5)))
    out = _mlp_call(_dec_body, [(lV, 0)], dec, [_sd((NPAD, 8), _F32)])
    return out[:N_NODES, :3]


# speedup vs baseline: 2.5265x; 1.0886x over previous
import functools

import jax
import jax.numpy as jnp
from jax import lax
from jax.experimental import pallas as pl
from jax.experimental.pallas import tpu as pltpu
from jax.experimental.pallas import tpu_sc as plsc

N_NODES = 10000
NPAD = 10240
N_EDGES = 320000
H = 128
ROW_TILE = 512
IDX_ROWS = N_EDGES // 128
GSLAB = 2
N_GSLABS = IDX_ROWS // GSLAB
HP = H // 2
SC_SLAB = 2
N_SC_SLABS = IDX_ROWS // SC_SLAB
NW = 32
E_CHUNKS = (160256, 159744)

_F32 = jnp.float32


def _dot(x, w):
    return jnp.dot(x.astype(jnp.bfloat16), w, preferred_element_type=_F32)


def _leaky(x):
    return jnp.where(x >= 0, x, 0.01 * x)


def _layernorm(x, g, b):
    mu = jnp.mean(x, axis=-1, keepdims=True)
    xc = x - mu
    var = jnp.mean(xc * xc, axis=-1, keepdims=True)
    return xc * lax.rsqrt(var + 1e-5) * g + b


def _tail3(x, w1, b1, w2, b2, w3, b3, g, bln):
    x = _leaky(x)
    x = _dot(x, w1[...]) + b1[...]
    x = _leaky(x)
    x = _dot(x, w2[...]) + b2[...]
    x = _leaky(x)
    x = _dot(x, w3[...]) + b3[...]
    return _layernorm(x, g[...], bln[...])


def _enc_body(x_ref, w0, b0, w1, b1, w2, b2, w3, b3, g, bln, o_ref):
    x = _dot(x_ref[...], w0[...]) + b0[...]
    o_ref[...] = _tail3(x, w1, b1, w2, b2, w3, b3, g, bln)


def _unpack2(xi):
    lo = lax.bitcast_convert_type(xi << 16, _F32)
    hi = lax.bitcast_convert_type(xi & jnp.int32(-65536), _F32)
    return lo, hi


def _enc_bf_body(x_ref, w0, b0, w1, b1, w2, b2, w3, b3, g, bln, o_ref):
    x = _dot(x_ref[...], w0[...]) + b0[...]
    o_ref[...] = _tail3(x, w1, b1, w2, b2, w3, b3, g, bln).astype(jnp.bfloat16)


def _enc_dual_body(x_ref, w0, b0, w1, b1, w2, b2, w3, b3, g, bln, o_ref, ob_ref):
    x = _dot(x_ref[...], w0[...]) + b0[...]
    y = _tail3(x, w1, b1, w2, b2, w3, b3, g, bln)
    o_ref[...] = y
    ob_ref[...] = y.astype(jnp.bfloat16)


def _msg_body(vd, vs, le, wde, wdo, wse, wso, wle, b0,
              w1, b1, w2, b2, w3, b3, g, bln, msg_o, le_o):
    de, do = _unpack2(vd[...])
    se, so = _unpack2(vs[...])
    x = (_dot(de, wde[...]) + _dot(do, wdo[...])
         + _dot(se, wse[...]) + _dot(so, wso[...])
         + _dot(le[...], wle[...]) + b0[...])
    m = _tail3(x, w1, b1, w2, b2, w3, b3, g, bln)
    msg_o[...] = m
    le_o[...] = (le[...].astype(_F32) + m).astype(jnp.bfloat16)


def _msg_f32_body(vd, vs, le, w0, b0, w1, b1, w2, b2, w3, b3, g, bln,
                  msg_o, le_o):
    x = (_dot(vd[...], w0[0:H, :]) + _dot(vs[...], w0[H:2 * H, :])
         + _dot(le[...], w0[2 * H:3 * H, :]) + b0[...])
    m = _tail3(x, w1, b1, w2, b2, w3, b3, g, bln)
    msg_o[...] = m
    le_o[...] = (le[...].astype(_F32) + m).astype(jnp.bfloat16)


def _upd_body(a0, a1, a2, a3, v, w0, b0, w1, b1, w2, b2, w3, b3, g, bln,
              o_ref, ob_ref):
    agg = a0[...] + a1[...] + a2[...] + a3[...]
    x = _dot(agg, w0[0:H, :]) + _dot(v[...], w0[H:2 * H, :]) + b0[...]
    y = v[...] + _tail3(x, w1, b1, w2, b2, w3, b3, g, bln)
    o_ref[...] = y
    ob_ref[...] = y.astype(jnp.bfloat16)


def _dec_body(x_ref, w0, b0, w1, b1, w2, b2, w3, b3, o_ref):
    x = _leaky(_dot(x_ref[...], w0[...]) + b0[...])
    x = _leaky(_dot(x, w1[...]) + b1[...])
    x = _leaky(_dot(x, w2[...]) + b2[...])
    o_ref[...] = _dot(x, w3[...]) + b3[...]


def _mlp_call(body, parts, weights, outs):
    n_rows = outs[0].shape[0]
    grid = (n_rows // ROW_TILE,)

    def _row_spec(ncols, off):
        return pl.BlockSpec((ROW_TILE, ncols), lambda i, o=off: (i + o, 0))

    in_specs = ([_row_spec(p.shape[1], off) for p, off in parts]
                + [pl.BlockSpec(w.shape, lambda i: (0, 0)) for w in weights])
    out_specs = [_row_spec(o.shape[1], 0) for o in outs]
    single = len(outs) == 1
    res = pl.pallas_call(
        body,
        grid=grid,
        in_specs=in_specs,
        out_specs=out_specs[0] if single else out_specs,
        out_shape=outs[0] if single else outs,
        compiler_params=pltpu.CompilerParams(
            dimension_semantics=("parallel",)),
    )(*[p for p, _ in parts], *weights)
    return res


def _prep_mlp(p):
    ws = []
    for l in p["linears"]:
        ws.append(l["W"].T.astype(jnp.bfloat16))
        ws.append(l["b"].reshape(1, -1).astype(_F32))
    if "ln" in p:
        ws.append(p["ln"]["g"].reshape(1, -1).astype(_F32))
        ws.append(p["ln"]["b"].reshape(1, -1).astype(_F32))
    return ws


def _prep_msg(p):
    ws = _prep_mlp(p)
    w0 = ws[0]
    wd, wsr, wle = w0[0:H], w0[H:2 * H], w0[2 * H:3 * H]
    return [wd[0::2], wd[1::2], wsr[0::2], wsr[1::2], wle] + ws[1:]


def _sc_mesh():
    return plsc.VectorSubcoreMesh(core_axis_name="c", subcore_axis_name="s")


def _sc_gather_pair(table, dst1, src1, n_edges):
    SLAB_E = GSLAB * 128

    @functools.partial(
        pl.kernel,
        mesh=_sc_mesh(),
        out_type=[jax.ShapeDtypeStruct((n_edges, H), _F32),
                  jax.ShapeDtypeStruct((n_edges, H), _F32)],
        scratch_types=[pltpu.VMEM((SLAB_E,), jnp.int32),
                       pltpu.VMEM((SLAB_E,), jnp.int32),
                       pltpu.VMEM((SLAB_E, H), _F32),
                       pltpu.VMEM((SLAB_E, H), _F32),
                       pltpu.SemaphoreType.DMA,
                       pltpu.SemaphoreType.DMA,
                       pltpu.SemaphoreType.DMA,
                       pltpu.SemaphoreType.DMA],
    )
    def gk(table_hbm, d_hbm, s_hbm, od_hbm, os_hbm,
           idx_d, idx_s, rows_d, rows_s, sem_d, sem_s, sem_od, sem_os):
        wid = lax.axis_index("s") * 2 + lax.axis_index("c")

        @pl.loop(wid, n_edges // SLAB_E, step=NW)
        def _(k):
            base = k * SLAB_E
            pltpu.sync_copy(d_hbm.at[pl.ds(base, SLAB_E)], idx_d)
            gd = [pltpu.async_copy(table_hbm.at[idx_d.at[pl.ds(j * 128, 128)]],
                                   rows_d.at[pl.ds(j * 128, 128)], sem_d)
                  for j in range(GSLAB)]
            pltpu.sync_copy(s_hbm.at[pl.ds(base, SLAB_E)], idx_s)
            gs = [pltpu.async_copy(table_hbm.at[idx_s.at[pl.ds(j * 128, 128)]],
                                   rows_s.at[pl.ds(j * 128, 128)], sem_s)
                  for j in range(GSLAB)]
            for cp in gd:
                cp.wait()
            od = pltpu.async_copy(rows_d, od_hbm.at[pl.ds(base, SLAB_E)],
                                  sem_od)
            for cp in gs:
                cp.wait()
            os_ = pltpu.async_copy(rows_s, os_hbm.at[pl.ds(base, SLAB_E)],
                                   sem_os)
            od.wait()
            os_.wait()

    return gk(table, dst1, src1)


def _sc_scatter_add(msg, dst2d, zeros, n_edges):
    @functools.partial(
        pl.kernel,
        mesh=_sc_mesh(),
        out_type=jax.ShapeDtypeStruct((2 * NPAD, H), _F32),
        scratch_types=[pltpu.VMEM((SC_SLAB, 128), jnp.int32),
                       pltpu.VMEM((SC_SLAB * 128, H), _F32),
                       pltpu.VMEM_SHARED((NPAD, H), _F32)],
    )
    def sk(msg_hbm, d_hbm, z_hbm, out_hbm, idx_v, rows_v, acc):
        c = lax.axis_index("c")
        s = lax.axis_index("s")
        wid = s * 2 + c
        rows_per = NPAD // 16
        pltpu.sync_copy(z_hbm.at[pl.ds(s * rows_per, rows_per)],
                        acc.at[pl.ds(s * rows_per, rows_per)])
        plsc.subcore_barrier()

        @pl.loop(wid, n_edges // (SC_SLAB * 128), step=NW)
        def _(k):
            pltpu.sync_copy(d_hbm.at[pl.ds(k * SC_SLAB, SC_SLAB)], idx_v)
            pltpu.sync_copy(msg_hbm.at[pl.ds(k * SC_SLAB * 128, SC_SLAB * 128)],
                            rows_v)
            for j in range(SC_SLAB):
                pltpu.sync_copy(rows_v.at[pl.ds(j * 128, 128)],
                                acc.at[idx_v.at[j]], add=True)

        plsc.subcore_barrier()
        pltpu.sync_copy(acc.at[pl.ds(s * rows_per, rows_per)],
                        out_hbm.at[pl.ds(c * NPAD + s * rows_per, rows_per)])

    return sk(msg, dst2d, zeros)


def kernel(V, E, edge_index, params):
    e0 = E_CHUNKS[0]
    src1 = [edge_index[0][:e0], edge_index[0][e0:]]
    dst1 = [edge_index[1][:e0], edge_index[1][e0:]]
    dst2d = [d.reshape(-1, 128) for d in dst1]
    Vp = jnp.pad(V, ((0, NPAD - N_NODES), (0, 0)))
    zeros = jnp.zeros((NPAD, H), _F32)

    _sd = jax.ShapeDtypeStruct
    bf16 = jnp.bfloat16
    lV, lVb = _mlp_call(_enc_dual_body, [(Vp, 0)], _prep_mlp(params["enc_V"]),
                        [_sd((NPAD, H), _F32), _sd((NPAD, H), bf16)])
    lE = _mlp_call(_enc_bf_body, [(E, 0)], _prep_mlp(params["enc_E"]),
                   [_sd((N_EDGES, H), bf16)])

    lE = [lE, None]
    le_off = [0, E_CHUNKS[0] // ROW_TILE]
    for p in params["procs"]:
        wm = _prep_mlp(p["msg"])
        aggs, new_lE = [], [None, None]
        for c in (0, 1):
            ec = E_CHUNKS[c]
            Vd, Vs = _sc_gather_pair(lV, dst1[c], src1[c], ec)
            le_arr = lE[0] if lE[1] is None else lE[c]
            off = le_off[c] if lE[1] is None else 0
            msg, new_lE[c] = _mlp_call(
                _msg_f32_body, [(Vd, 0), (Vs, 0), (le_arr, off)], wm,
                [_sd((ec, H), _F32), _sd((ec, H), bf16)])
            aggs.append(_sc_scatter_add(msg, dst2d[c], zeros, ec))
        lE = new_lE
        npb = NPAD // ROW_TILE
        lV, lVb = _mlp_call(_upd_body,
                            [(aggs[0], 0), (aggs[0], npb),
                             (aggs[1], 0), (aggs[1], npb), (lV, 0)],
                            _prep_mlp(p["upd"]),
                            [_sd((NPAD, H), _F32), _sd((NPAD, H), bf16)])

    dec = _prep_mlp(params["dec"])
    dec[6] = jnp.pad(dec[6], ((0, 0), (0, 5)))
    dec[7] = jnp.pad(dec[7], ((0, 0), (0, 5)))
    out = _mlp_call(_dec_body, [(lV, 0)], dec, [_sd((NPAD, 8), _F32)])
    return out[:N_NODES, :3]


# speedup vs baseline: 2.5287x; 1.0009x over previous
import functools

import jax
import jax.numpy as jnp
from jax import lax
from jax.experimental import pallas as pl
from jax.experimental.pallas import tpu as pltpu
from jax.experimental.pallas import tpu_sc as plsc

N_NODES = 10000
NPAD = 10240
N_EDGES = 320000
H = 128
ROW_TILE = 512
IDX_ROWS = N_EDGES // 128
GSLAB = 2
N_GSLABS = IDX_ROWS // GSLAB
HP = H // 2
SC_SLAB = 2
N_SC_SLABS = IDX_ROWS // SC_SLAB
NW = 32
E_CHUNKS = (160256, 159744)

_F32 = jnp.float32


def _dot(x, w):
    return jnp.dot(x.astype(jnp.bfloat16), w, preferred_element_type=_F32)


def _leaky(x):
    return jnp.where(x >= 0, x, 0.01 * x)


def _layernorm(x, g, b):
    mu = jnp.mean(x, axis=-1, keepdims=True)
    xc = x - mu
    var = jnp.mean(xc * xc, axis=-1, keepdims=True)
    return xc * lax.rsqrt(var + 1e-5) * g + b


def _tail3(x, w1, b1, w2, b2, w3, b3, g, bln):
    x = _leaky(x)
    x = _dot(x, w1[...]) + b1[...]
    x = _leaky(x)
    x = _dot(x, w2[...]) + b2[...]
    x = _leaky(x)
    x = _dot(x, w3[...]) + b3[...]
    return _layernorm(x, g[...], bln[...])


def _enc_body(x_ref, w0, b0, w1, b1, w2, b2, w3, b3, g, bln, o_ref):
    x = _dot(x_ref[...], w0[...]) + b0[...]
    o_ref[...] = _tail3(x, w1, b1, w2, b2, w3, b3, g, bln)


def _unpack2(xi):
    lo = lax.bitcast_convert_type(xi << 16, _F32)
    hi = lax.bitcast_convert_type(xi & jnp.int32(-65536), _F32)
    return lo, hi


def _enc_bf_body(x_ref, w0, b0, w1, b1, w2, b2, w3, b3, g, bln, o_ref):
    x = _dot(x_ref[...], w0[...]) + b0[...]
    o_ref[...] = _tail3(x, w1, b1, w2, b2, w3, b3, g, bln).astype(jnp.bfloat16)


def _enc_dual_body(x_ref, w0, b0, w1, b1, w2, b2, w3, b3, g, bln, o_ref, ob_ref):
    x = _dot(x_ref[...], w0[...]) + b0[...]
    y = _tail3(x, w1, b1, w2, b2, w3, b3, g, bln)
    o_ref[...] = y
    ob_ref[...] = y.astype(jnp.bfloat16)


def _msg_body(vd, vs, le, wde, wdo, wse, wso, wle, b0,
              w1, b1, w2, b2, w3, b3, g, bln, msg_o, le_o):
    de, do = _unpack2(vd[...])
    se, so = _unpack2(vs[...])
    x = (_dot(de, wde[...]) + _dot(do, wdo[...])
         + _dot(se, wse[...]) + _dot(so, wso[...])
         + _dot(le[...], wle[...]) + b0[...])
    m = _tail3(x, w1, b1, w2, b2, w3, b3, g, bln)
    msg_o[...] = m
    le_o[...] = (le[...].astype(_F32) + m).astype(jnp.bfloat16)


def _msg_f32_body(vd, vs, le, w0, b0, w1, b1, w2, b2, w3, b3, g, bln,
                  msg_o, le_o):
    x = (_dot(vd[...], w0[0:H, :]) + _dot(vs[...], w0[H:2 * H, :])
         + _dot(le[...], w0[2 * H:3 * H, :]) + b0[...])
    m = _tail3(x, w1, b1, w2, b2, w3, b3, g, bln)
    msg_o[...] = m
    le_o[...] = (le[...].astype(_F32) + m).astype(jnp.bfloat16)


def _upd_body(a0, a1, a2, a3, v, w0, b0, w1, b1, w2, b2, w3, b3, g, bln,
              o_ref, ob_ref):
    agg = a0[...] + a1[...] + a2[...] + a3[...]
    x = _dot(agg, w0[0:H, :]) + _dot(v[...], w0[H:2 * H, :]) + b0[...]
    y = v[...] + _tail3(x, w1, b1, w2, b2, w3, b3, g, bln)
    o_ref[...] = y
    ob_ref[...] = y.astype(jnp.bfloat16)


def _dec_body(x_ref, w0, b0, w1, b1, w2, b2, w3, b3, o_ref):
    x = _leaky(_dot(x_ref[...], w0[...]) + b0[...])
    x = _leaky(_dot(x, w1[...]) + b1[...])
    x = _leaky(_dot(x, w2[...]) + b2[...])
    o_ref[...] = _dot(x, w3[...]) + b3[...]


def _mlp_call(body, parts, weights, outs):
    n_rows = outs[0].shape[0]
    grid = (n_rows // ROW_TILE,)

    def _row_spec(ncols, off):
        return pl.BlockSpec((ROW_TILE, ncols), lambda i, o=off: (i + o, 0))

    in_specs = ([_row_spec(p.shape[1], off) for p, off in parts]
                + [pl.BlockSpec(w.shape, lambda i: (0, 0)) for w in weights])
    out_specs = [_row_spec(o.shape[1], 0) for o in outs]
    single = len(outs) == 1
    res = pl.pallas_call(
        body,
        grid=grid,
        in_specs=in_specs,
        out_specs=out_specs[0] if single else out_specs,
        out_shape=outs[0] if single else outs,
        compiler_params=pltpu.CompilerParams(
            dimension_semantics=("parallel",)),
    )(*[p for p, _ in parts], *weights)
    return res


def _prep_mlp(p):
    ws = []
    for l in p["linears"]:
        ws.append(l["W"].T.astype(jnp.bfloat16))
        ws.append(l["b"].reshape(1, -1).astype(_F32))
    if "ln" in p:
        ws.append(p["ln"]["g"].reshape(1, -1).astype(_F32))
        ws.append(p["ln"]["b"].reshape(1, -1).astype(_F32))
    return ws


def _prep_msg(p):
    ws = _prep_mlp(p)
    w0 = ws[0]
    wd, wsr, wle = w0[0:H], w0[H:2 * H], w0[2 * H:3 * H]
    return [wd[0::2], wd[1::2], wsr[0::2], wsr[1::2], wle] + ws[1:]


def _sc_mesh():
    return plsc.VectorSubcoreMesh(core_axis_name="c", subcore_axis_name="s")


def _sc_gather_pair(table, dst1, src1, n_edges):
    SLAB_E = GSLAB * 128

    @functools.partial(
        pl.kernel,
        mesh=_sc_mesh(),
        out_type=[jax.ShapeDtypeStruct((n_edges, H), _F32),
                  jax.ShapeDtypeStruct((n_edges, H), _F32)],
        scratch_types=[pltpu.VMEM((SLAB_E,), jnp.int32),
                       pltpu.VMEM((SLAB_E,), jnp.int32),
                       pltpu.VMEM((SLAB_E, H), _F32),
                       pltpu.VMEM((SLAB_E, H), _F32),
                       pltpu.SemaphoreType.DMA,
                       pltpu.SemaphoreType.DMA,
                       pltpu.SemaphoreType.DMA,
                       pltpu.SemaphoreType.DMA],
    )
    def gk(table_hbm, d_hbm, s_hbm, od_hbm, os_hbm,
           idx_d, idx_s, rows_d, rows_s, sem_d, sem_s, sem_od, sem_os):
        wid = lax.axis_index("s") * 2 + lax.axis_index("c")
        nslab = n_edges // SLAB_E
        pltpu.sync_copy(d_hbm.at[pl.ds(wid * SLAB_E, SLAB_E)], idx_d)
        pltpu.sync_copy(s_hbm.at[pl.ds(wid * SLAB_E, SLAB_E)], idx_s)

        @pl.loop(wid, nslab, step=NW)
        def _(k):
            base = k * SLAB_E

            @pl.when(k != wid)
            def _():
                pltpu.make_async_copy(
                    rows_d, od_hbm.at[pl.ds(base - NW * SLAB_E, SLAB_E)],
                    sem_od).wait()
                pltpu.make_async_copy(
                    rows_s, os_hbm.at[pl.ds(base - NW * SLAB_E, SLAB_E)],
                    sem_os).wait()

            gd = [pltpu.async_copy(table_hbm.at[idx_d.at[pl.ds(j * 128, 128)]],
                                   rows_d.at[pl.ds(j * 128, 128)], sem_d)
                  for j in range(GSLAB)]
            gs = [pltpu.async_copy(table_hbm.at[idx_s.at[pl.ds(j * 128, 128)]],
                                   rows_s.at[pl.ds(j * 128, 128)], sem_s)
                  for j in range(GSLAB)]
            for cp in gd:
                cp.wait()
            for cp in gs:
                cp.wait()
            pltpu.async_copy(rows_d, od_hbm.at[pl.ds(base, SLAB_E)], sem_od)
            pltpu.async_copy(rows_s, os_hbm.at[pl.ds(base, SLAB_E)], sem_os)

            @pl.when(k + NW < nslab)
            def _():
                pltpu.sync_copy(d_hbm.at[pl.ds(base + NW * SLAB_E, SLAB_E)],
                                idx_d)
                pltpu.sync_copy(s_hbm.at[pl.ds(base + NW * SLAB_E, SLAB_E)],
                                idx_s)

        pltpu.make_async_copy(rows_d, od_hbm.at[pl.ds(0, SLAB_E)],
                              sem_od).wait()
        pltpu.make_async_copy(rows_s, os_hbm.at[pl.ds(0, SLAB_E)],
                              sem_os).wait()

    return gk(table, dst1, src1)


def _sc_scatter_add(msg, dst2d, zeros, n_edges):
    @functools.partial(
        pl.kernel,
        mesh=_sc_mesh(),
        out_type=jax.ShapeDtypeStruct((2 * NPAD, H), _F32),
        scratch_types=[pltpu.VMEM((SC_SLAB, 128), jnp.int32),
                       pltpu.VMEM((SC_SLAB * 128, H), _F32),
                       pltpu.VMEM_SHARED((NPAD, H), _F32)],
    )
    def sk(msg_hbm, d_hbm, z_hbm, out_hbm, idx_v, rows_v, acc):
        c = lax.axis_index("c")
        s = lax.axis_index("s")
        wid = s * 2 + c
        rows_per = NPAD // 16
        pltpu.sync_copy(z_hbm.at[pl.ds(s * rows_per, rows_per)],
                        acc.at[pl.ds(s * rows_per, rows_per)])
        plsc.subcore_barrier()

        @pl.loop(wid, n_edges // (SC_SLAB * 128), step=NW)
        def _(k):
            pltpu.sync_copy(d_hbm.at[pl.ds(k * SC_SLAB, SC_SLAB)], idx_v)
            pltpu.sync_copy(msg_hbm.at[pl.ds(k * SC_SLAB * 128, SC_SLAB * 128)],
                            rows_v)
            for j in range(SC_SLAB):
                pltpu.sync_copy(rows_v.at[pl.ds(j * 128, 128)],
                                acc.at[idx_v.at[j]], add=True)

        plsc.subcore_barrier()
        pltpu.sync_copy(acc.at[pl.ds(s * rows_per, rows_per)],
                        out_hbm.at[pl.ds(c * NPAD + s * rows_per, rows_per)])

    return sk(msg, dst2d, zeros)


def kernel(V, E, edge_index, params):
    e0 = E_CHUNKS[0]
    src1 = [edge_index[0][:e0], edge_index[0][e0:]]
    dst1 = [edge_index[1][:e0], edge_index[1][e0:]]
    dst2d = [d.reshape(-1, 128) for d in dst1]
    Vp = jnp.pad(V, ((0, NPAD - N_NODES), (0, 0)))
    zeros = jnp.zeros((NPAD, H), _F32)

    _sd = jax.ShapeDtypeStruct
    bf16 = jnp.bfloat16
    lV, lVb = _mlp_call(_enc_dual_body, [(Vp, 0)], _prep_mlp(params["enc_V"]),
                        [_sd((NPAD, H), _F32), _sd((NPAD, H), bf16)])
    lE = _mlp_call(_enc_bf_body, [(E, 0)], _prep_mlp(params["enc_E"]),
                   [_sd((N_EDGES, H), bf16)])

    lE = [lE, None]
    le_off = [0, E_CHUNKS[0] // ROW_TILE]
    for p in params["procs"]:
        wm = _prep_mlp(p["msg"])
        aggs, new_lE = [], [None, None]
        for c in (0, 1):
            ec = E_CHUNKS[c]
            Vd, Vs = _sc_gather_pair(lV, dst1[c], src1[c], ec)
            le_arr = lE[0] if lE[1] is None else lE[c]
            off = le_off[c] if lE[1] is None else 0
            msg, new_lE[c] = _mlp_call(
                _msg_f32_body, [(Vd, 0), (Vs, 0), (le_arr, off)], wm,
                [_sd((ec, H), _F32), _sd((ec, H), bf16)])
            aggs.append(_sc_scatter_add(msg, dst2d[c], zeros, ec))
        lE = new_lE
        npb = NPAD // ROW_TILE
        lV, lVb = _mlp_call(_upd_body,
                            [(aggs[0], 0), (aggs[0], npb),
                             (aggs[1], 0), (aggs[1], npb), (lV, 0)],
                            _prep_mlp(p["upd"]),
                            [_sd((NPAD, H), _F32), _sd((NPAD, H), bf16)])

    dec = _prep_mlp(params["dec"])
    dec[6] = jnp.pad(dec[6], ((0, 0), (0, 5)))
    dec[7] = jnp.pad(dec[7], ((0, 0), (0, 5)))
    out = _mlp_call(_dec_body, [(lV, 0)], dec, [_sd((NPAD, 8), _F32)])
    return out[:N_NODES, :3]


# speedup vs baseline: 2.5950x; 1.0262x over previous
import functools

import jax
import jax.numpy as jnp
from jax import lax
from jax.experimental import pallas as pl
from jax.experimental.pallas import tpu as pltpu
from jax.experimental.pallas import tpu_sc as plsc

N_NODES = 10000
NPAD = 10240
N_EDGES = 320000
H = 128
ROW_TILE = 512
IDX_ROWS = N_EDGES // 128
GSLAB = 2
N_GSLABS = IDX_ROWS // GSLAB
HP = H // 2
SC_SLAB = 2
N_SC_SLABS = IDX_ROWS // SC_SLAB
NW = 32
E_CHUNKS = (160256, 159744)

_F32 = jnp.float32


def _dot(x, w):
    return jnp.dot(x.astype(_F32), w, preferred_element_type=_F32)


def _leaky(x):
    return jnp.where(x >= 0, x, 0.01 * x)


def _layernorm(x, g, b):
    mu = jnp.mean(x, axis=-1, keepdims=True)
    xc = x - mu
    var = jnp.mean(xc * xc, axis=-1, keepdims=True)
    return xc * lax.rsqrt(var + 1e-5) * g + b


def _tail3(x, w1, b1, w2, b2, w3, b3, g, bln):
    x = _leaky(x)
    x = _dot(x, w1[...]) + b1[...]
    x = _leaky(x)
    x = _dot(x, w2[...]) + b2[...]
    x = _leaky(x)
    x = _dot(x, w3[...]) + b3[...]
    return _layernorm(x, g[...], bln[...])


def _enc_body(x_ref, w0, b0, w1, b1, w2, b2, w3, b3, g, bln, o_ref):
    x = _dot(x_ref[...], w0[...]) + b0[...]
    o_ref[...] = _tail3(x, w1, b1, w2, b2, w3, b3, g, bln)


def _unpack2(xi):
    lo = lax.bitcast_convert_type(xi << 16, _F32)
    hi = lax.bitcast_convert_type(xi & jnp.int32(-65536), _F32)
    return lo, hi


def _enc_bf_body(x_ref, w0, b0, w1, b1, w2, b2, w3, b3, g, bln, o_ref):
    x = _dot(x_ref[...], w0[...]) + b0[...]
    o_ref[...] = _tail3(x, w1, b1, w2, b2, w3, b3, g, bln).astype(jnp.bfloat16)


def _enc_dual_body(x_ref, w0, b0, w1, b1, w2, b2, w3, b3, g, bln, o_ref, ob_ref):
    x = _dot(x_ref[...], w0[...]) + b0[...]
    y = _tail3(x, w1, b1, w2, b2, w3, b3, g, bln)
    o_ref[...] = y
    ob_ref[...] = y.astype(jnp.bfloat16)


def _msg_body(vd, vs, le, wde, wdo, wse, wso, wle, b0,
              w1, b1, w2, b2, w3, b3, g, bln, msg_o, le_o):
    de, do = _unpack2(vd[...])
    se, so = _unpack2(vs[...])
    x = (_dot(de, wde[...]) + _dot(do, wdo[...])
         + _dot(se, wse[...]) + _dot(so, wso[...])
         + _dot(le[...], wle[...]) + b0[...])
    m = _tail3(x, w1, b1, w2, b2, w3, b3, g, bln)
    msg_o[...] = m
    le_o[...] = (le[...].astype(_F32) + m).astype(jnp.bfloat16)


def _msg_f32_body(vd, vs, le, w0, b0, w1, b1, w2, b2, w3, b3, g, bln,
                  msg_o, le_o):
    x = (_dot(vd[...], w0[0:H, :]) + _dot(vs[...], w0[H:2 * H, :])
         + _dot(le[...], w0[2 * H:3 * H, :]) + b0[...])
    m = _tail3(x, w1, b1, w2, b2, w3, b3, g, bln)
    msg_o[...] = m
    le_o[...] = (le[...].astype(_F32) + m).astype(jnp.bfloat16)


def _upd_body(a0, a1, a2, a3, v, w0, b0, w1, b1, w2, b2, w3, b3, g, bln,
              o_ref, ob_ref):
    agg = a0[...] + a1[...] + a2[...] + a3[...]
    x = _dot(agg, w0[0:H, :]) + _dot(v[...], w0[H:2 * H, :]) + b0[...]
    y = v[...] + _tail3(x, w1, b1, w2, b2, w3, b3, g, bln)
    o_ref[...] = y
    ob_ref[...] = y.astype(jnp.bfloat16)


def _dec_body(x_ref, w0, b0, w1, b1, w2, b2, w3, b3, o_ref):
    x = _leaky(_dot(x_ref[...], w0[...]) + b0[...])
    x = _leaky(_dot(x, w1[...]) + b1[...])
    x = _leaky(_dot(x, w2[...]) + b2[...])
    o_ref[...] = _dot(x, w3[...]) + b3[...]


def _mlp_call(body, parts, weights, outs):
    n_rows = outs[0].shape[0]
    grid = (n_rows // ROW_TILE,)

    def _row_spec(ncols, off):
        return pl.BlockSpec((ROW_TILE, ncols), lambda i, o=off: (i + o, 0))

    in_specs = ([_row_spec(p.shape[1], off) for p, off in parts]
                + [pl.BlockSpec(w.shape, lambda i: (0, 0)) for w in weights])
    out_specs = [_row_spec(o.shape[1], 0) for o in outs]
    single = len(outs) == 1
    res = pl.pallas_call(
        body,
        grid=grid,
        in_specs=in_specs,
        out_specs=out_specs[0] if single else out_specs,
        out_shape=outs[0] if single else outs,
        compiler_params=pltpu.CompilerParams(
            dimension_semantics=("parallel",)),
    )(*[p for p, _ in parts], *weights)
    return res


def _prep_mlp(p):
    ws = []
    for l in p["linears"]:
        ws.append(l["W"].T.astype(_F32))
        ws.append(l["b"].reshape(1, -1).astype(_F32))
    if "ln" in p:
        ws.append(p["ln"]["g"].reshape(1, -1).astype(_F32))
        ws.append(p["ln"]["b"].reshape(1, -1).astype(_F32))
    return ws


def _prep_msg(p):
    ws = _prep_mlp(p)
    w0 = ws[0]
    wd, wsr, wle = w0[0:H], w0[H:2 * H], w0[2 * H:3 * H]
    return [wd[0::2], wd[1::2], wsr[0::2], wsr[1::2], wle] + ws[1:]


def _sc_mesh():
    return plsc.VectorSubcoreMesh(core_axis_name="c", subcore_axis_name="s")


def _sc_gather_pair(table, dst1, src1, n_edges):
    SLAB_E = GSLAB * 128

    @functools.partial(
        pl.kernel,
        mesh=_sc_mesh(),
        out_type=[jax.ShapeDtypeStruct((n_edges, H), _F32),
                  jax.ShapeDtypeStruct((n_edges, H), _F32)],
        scratch_types=[pltpu.VMEM((SLAB_E,), jnp.int32),
                       pltpu.VMEM((SLAB_E,), jnp.int32),
                       pltpu.VMEM((SLAB_E, H), _F32),
                       pltpu.VMEM((SLAB_E, H), _F32),
                       pltpu.SemaphoreType.DMA,
                       pltpu.SemaphoreType.DMA,
                       pltpu.SemaphoreType.DMA,
                       pltpu.SemaphoreType.DMA],
    )
    def gk(table_hbm, d_hbm, s_hbm, od_hbm, os_hbm,
           idx_d, idx_s, rows_d, rows_s, sem_d, sem_s, sem_od, sem_os):
        wid = lax.axis_index("s") * 2 + lax.axis_index("c")
        nslab = n_edges // SLAB_E
        pltpu.sync_copy(d_hbm.at[pl.ds(wid * SLAB_E, SLAB_E)], idx_d)
        pltpu.sync_copy(s_hbm.at[pl.ds(wid * SLAB_E, SLAB_E)], idx_s)

        @pl.loop(wid, nslab, step=NW)
        def _(k):
            base = k * SLAB_E

            @pl.when(k != wid)
            def _():
                pltpu.make_async_copy(
                    rows_d, od_hbm.at[pl.ds(base - NW * SLAB_E, SLAB_E)],
                    sem_od).wait()
                pltpu.make_async_copy(
                    rows_s, os_hbm.at[pl.ds(base - NW * SLAB_E, SLAB_E)],
                    sem_os).wait()

            gd = [pltpu.async_copy(table_hbm.at[idx_d.at[pl.ds(j * 128, 128)]],
                                   rows_d.at[pl.ds(j * 128, 128)], sem_d)
                  for j in range(GSLAB)]
            gs = [pltpu.async_copy(table_hbm.at[idx_s.at[pl.ds(j * 128, 128)]],
                                   rows_s.at[pl.ds(j * 128, 128)], sem_s)
                  for j in range(GSLAB)]
            for cp in gd:
                cp.wait()
            for cp in gs:
                cp.wait()
            pltpu.async_copy(rows_d, od_hbm.at[pl.ds(base, SLAB_E)], sem_od)
            pltpu.async_copy(rows_s, os_hbm.at[pl.ds(base, SLAB_E)], sem_os)

            @pl.when(k + NW < nslab)
            def _():
                pltpu.sync_copy(d_hbm.at[pl.ds(base + NW * SLAB_E, SLAB_E)],
                                idx_d)
                pltpu.sync_copy(s_hbm.at[pl.ds(base + NW * SLAB_E, SLAB_E)],
                                idx_s)

        pltpu.make_async_copy(rows_d, od_hbm.at[pl.ds(0, SLAB_E)],
                              sem_od).wait()
        pltpu.make_async_copy(rows_s, os_hbm.at[pl.ds(0, SLAB_E)],
                              sem_os).wait()

    return gk(table, dst1, src1)


def _sc_scatter_add(msg, dst2d, zeros, n_edges):
    nslab = n_edges // 128

    @functools.partial(
        pl.kernel,
        mesh=_sc_mesh(),
        out_type=jax.ShapeDtypeStruct((2 * NPAD, H), _F32),
        scratch_types=[pltpu.VMEM((1, 128), jnp.int32),
                       pltpu.VMEM((1, 128), jnp.int32),
                       pltpu.VMEM((128, H), _F32),
                       pltpu.VMEM((128, H), _F32),
                       pltpu.VMEM_SHARED((NPAD, H), _F32),
                       pltpu.SemaphoreType.DMA,
                       pltpu.SemaphoreType.DMA],
    )
    def sk(msg_hbm, d_hbm, z_hbm, out_hbm, idx_a, idx_b, rows_a, rows_b,
           acc, sem_a, sem_b):
        c = lax.axis_index("c")
        s = lax.axis_index("s")
        wid = s * 2 + c
        rows_per = NPAD // 16
        pltpu.sync_copy(z_hbm.at[pl.ds(s * rows_per, rows_per)],
                        acc.at[pl.ds(s * rows_per, rows_per)])
        plsc.subcore_barrier()

        def load(k, idx_v, rows_v, sem):
            pltpu.async_copy(d_hbm.at[pl.ds(k, 1)], idx_v, sem)
            pltpu.async_copy(msg_hbm.at[pl.ds(k * 128, 128)], rows_v, sem)

        def drain(k, idx_v, rows_v, sem):
            pltpu.make_async_copy(d_hbm.at[pl.ds(k, 1)], idx_v, sem).wait()
            pltpu.make_async_copy(msg_hbm.at[pl.ds(k * 128, 128)], rows_v,
                                  sem).wait()

        load(wid, idx_a, rows_a, sem_a)

        @pl.loop(wid, nslab, step=2 * NW)
        def _(k):
            kb = k + NW

            @pl.when(kb < nslab)
            def _():
                load(kb, idx_b, rows_b, sem_b)

            drain(k, idx_a, rows_a, sem_a)
            pltpu.sync_copy(rows_a, acc.at[idx_a.at[0]], add=True)

            @pl.when(k + 2 * NW < nslab)
            def _():
                load(k + 2 * NW, idx_a, rows_a, sem_a)

            @pl.when(kb < nslab)
            def _():
                drain(kb, idx_b, rows_b, sem_b)
                pltpu.sync_copy(rows_b, acc.at[idx_b.at[0]], add=True)

        plsc.subcore_barrier()
        pltpu.sync_copy(acc.at[pl.ds(s * rows_per, rows_per)],
                        out_hbm.at[pl.ds(c * NPAD + s * rows_per, rows_per)])

    return sk(msg, dst2d, zeros)


def kernel(V, E, edge_index, params):
    e0 = E_CHUNKS[0]
    src1 = [edge_index[0][:e0], edge_index[0][e0:]]
    dst1 = [edge_index[1][:e0], edge_index[1][e0:]]
    dst2d = [d.reshape(-1, 128) for d in dst1]
    Vp = jnp.pad(V, ((0, NPAD - N_NODES), (0, 0)))
    zeros = jnp.zeros((NPAD, H), _F32)

    _sd = jax.ShapeDtypeStruct
    bf16 = jnp.bfloat16
    lV, lVb = _mlp_call(_enc_dual_body, [(Vp, 0)], _prep_mlp(params["enc_V"]),
                        [_sd((NPAD, H), _F32), _sd((NPAD, H), bf16)])
    lE = _mlp_call(_enc_bf_body, [(E, 0)], _prep_mlp(params["enc_E"]),
                   [_sd((N_EDGES, H), bf16)])

    lE = [lE, None]
    le_off = [0, E_CHUNKS[0] // ROW_TILE]
    for p in params["procs"]:
        wm = _prep_mlp(p["msg"])
        aggs, new_lE = [], [None, None]
        for c in (0, 1):
            ec = E_CHUNKS[c]
            Vd, Vs = _sc_gather_pair(lV, dst1[c], src1[c], ec)
            le_arr = lE[0] if lE[1] is None else lE[c]
            off = le_off[c] if lE[1] is None else 0
            msg, new_lE[c] = _mlp_call(
                _msg_f32_body, [(Vd, 0), (Vs, 0), (le_arr, off)], wm,
                [_sd((ec, H), _F32), _sd((ec, H), bf16)])
            aggs.append(_sc_scatter_add(msg, dst2d[c], zeros, ec))
        lE = new_lE
        npb = NPAD // ROW_TILE
        lV, lVb = _mlp_call(_upd_body,
                            [(aggs[0], 0), (aggs[0], npb),
                             (aggs[1], 0), (aggs[1], npb), (lV, 0)],
                            _prep_mlp(p["upd"]),
                            [_sd((NPAD, H), _F32), _sd((NPAD, H), bf16)])

    dec = _prep_mlp(params["dec"])
    dec[6] = jnp.pad(dec[6], ((0, 0), (0, 5)))
    dec[7] = jnp.pad(dec[7], ((0, 0), (0, 5)))
    out = _mlp_call(_dec_body, [(lV, 0)], dec, [_sd((NPAD, 8), _F32)])
    return out[:N_NODES, :3]


# speedup vs baseline: 2.6728x; 1.0300x over previous
import functools

import jax
import jax.numpy as jnp
from jax import lax
from jax.experimental import pallas as pl
from jax.experimental.pallas import tpu as pltpu
from jax.experimental.pallas import tpu_sc as plsc

N_NODES = 10000
NPAD = 10240
N_EDGES = 320000
H = 128
ROW_TILE = 512
IDX_ROWS = N_EDGES // 128
GSLAB = 2
N_GSLABS = IDX_ROWS // GSLAB
HP = H // 2
SC_SLAB = 2
N_SC_SLABS = IDX_ROWS // SC_SLAB
NW = 32
E_CHUNKS = (80384, 79872, 79872, 79872)

_F32 = jnp.float32


def _dot(x, w):
    return jnp.dot(x.astype(_F32), w, preferred_element_type=_F32)


def _leaky(x):
    return jnp.where(x >= 0, x, 0.01 * x)


def _layernorm(x, g, b):
    mu = jnp.mean(x, axis=-1, keepdims=True)
    xc = x - mu
    var = jnp.mean(xc * xc, axis=-1, keepdims=True)
    return xc * lax.rsqrt(var + 1e-5) * g + b


def _tail3(x, w1, b1, w2, b2, w3, b3, g, bln):
    x = _leaky(x)
    x = _dot(x, w1[...]) + b1[...]
    x = _leaky(x)
    x = _dot(x, w2[...]) + b2[...]
    x = _leaky(x)
    x = _dot(x, w3[...]) + b3[...]
    return _layernorm(x, g[...], bln[...])


def _enc_body(x_ref, w0, b0, w1, b1, w2, b2, w3, b3, g, bln, o_ref):
    x = _dot(x_ref[...], w0[...]) + b0[...]
    o_ref[...] = _tail3(x, w1, b1, w2, b2, w3, b3, g, bln)


def _unpack2(xi):
    lo = lax.bitcast_convert_type(xi << 16, _F32)
    hi = lax.bitcast_convert_type(xi & jnp.int32(-65536), _F32)
    return lo, hi


def _enc_bf_body(x_ref, w0, b0, w1, b1, w2, b2, w3, b3, g, bln, o_ref):
    x = _dot(x_ref[...], w0[...]) + b0[...]
    o_ref[...] = _tail3(x, w1, b1, w2, b2, w3, b3, g, bln).astype(jnp.bfloat16)


def _enc_dual_body(x_ref, w0, b0, w1, b1, w2, b2, w3, b3, g, bln, o_ref, ob_ref):
    x = _dot(x_ref[...], w0[...]) + b0[...]
    y = _tail3(x, w1, b1, w2, b2, w3, b3, g, bln)
    o_ref[...] = y
    ob_ref[...] = y.astype(jnp.bfloat16)


def _msg_body(vd, vs, le, wde, wdo, wse, wso, wle, b0,
              w1, b1, w2, b2, w3, b3, g, bln, msg_o, le_o):
    de, do = _unpack2(vd[...])
    se, so = _unpack2(vs[...])
    x = (_dot(de, wde[...]) + _dot(do, wdo[...])
         + _dot(se, wse[...]) + _dot(so, wso[...])
         + _dot(le[...], wle[...]) + b0[...])
    m = _tail3(x, w1, b1, w2, b2, w3, b3, g, bln)
    msg_o[...] = m
    le_o[...] = (le[...].astype(_F32) + m).astype(jnp.bfloat16)


def _msg_f32_body(vd, vs, le, w0, b0, w1, b1, w2, b2, w3, b3, g, bln,
                  msg_o, le_o):
    x = (_dot(vd[...], w0[0:H, :]) + _dot(vs[...], w0[H:2 * H, :])
         + _dot(le[...], w0[2 * H:3 * H, :]) + b0[...])
    m = _tail3(x, w1, b1, w2, b2, w3, b3, g, bln)
    msg_o[...] = m
    le_o[...] = (le[...].astype(_F32) + m).astype(jnp.bfloat16)


def _make_upd_body(n_agg):
    def body(*refs):
        aggs = refs[:n_agg]
        v = refs[n_agg]
        w0, b0, w1, b1, w2, b2, w3, b3, g, bln = refs[n_agg + 1:n_agg + 11]
        o_ref, ob_ref = refs[n_agg + 11:]
        agg = aggs[0][...]
        for a in aggs[1:]:
            agg = agg + a[...]
        x = _dot(agg, w0[0:H, :]) + _dot(v[...], w0[H:2 * H, :]) + b0[...]
        y = v[...] + _tail3(x, w1, b1, w2, b2, w3, b3, g, bln)
        o_ref[...] = y
        ob_ref[...] = y.astype(jnp.bfloat16)
    return body


def _dec_body(x_ref, w0, b0, w1, b1, w2, b2, w3, b3, o_ref):
    x = _leaky(_dot(x_ref[...], w0[...]) + b0[...])
    x = _leaky(_dot(x, w1[...]) + b1[...])
    x = _leaky(_dot(x, w2[...]) + b2[...])
    o_ref[...] = _dot(x, w3[...]) + b3[...]


def _mlp_call(body, parts, weights, outs):
    n_rows = outs[0].shape[0]
    grid = (n_rows // ROW_TILE,)

    def _row_spec(ncols, off):
        return pl.BlockSpec((ROW_TILE, ncols), lambda i, o=off: (i + o, 0))

    in_specs = ([_row_spec(p.shape[1], off) for p, off in parts]
                + [pl.BlockSpec(w.shape, lambda i: (0, 0)) for w in weights])
    out_specs = [_row_spec(o.shape[1], 0) for o in outs]
    single = len(outs) == 1
    res = pl.pallas_call(
        body,
        grid=grid,
        in_specs=in_specs,
        out_specs=out_specs[0] if single else out_specs,
        out_shape=outs[0] if single else outs,
        compiler_params=pltpu.CompilerParams(
            dimension_semantics=("parallel",)),
    )(*[p for p, _ in parts], *weights)
    return res


def _prep_mlp(p):
    ws = []
    for l in p["linears"]:
        ws.append(l["W"].T.astype(_F32))
        ws.append(l["b"].reshape(1, -1).astype(_F32))
    if "ln" in p:
        ws.append(p["ln"]["g"].reshape(1, -1).astype(_F32))
        ws.append(p["ln"]["b"].reshape(1, -1).astype(_F32))
    return ws


def _prep_msg(p):
    ws = _prep_mlp(p)
    w0 = ws[0]
    wd, wsr, wle = w0[0:H], w0[H:2 * H], w0[2 * H:3 * H]
    return [wd[0::2], wd[1::2], wsr[0::2], wsr[1::2], wle] + ws[1:]


def _sc_mesh():
    return plsc.VectorSubcoreMesh(core_axis_name="c", subcore_axis_name="s")


def _sc_gather_pair(table, dst1, src1, n_edges):
    SLAB_E = GSLAB * 128

    @functools.partial(
        pl.kernel,
        mesh=_sc_mesh(),
        out_type=[jax.ShapeDtypeStruct((n_edges, H), _F32),
                  jax.ShapeDtypeStruct((n_edges, H), _F32)],
        scratch_types=[pltpu.VMEM((SLAB_E,), jnp.int32),
                       pltpu.VMEM((SLAB_E,), jnp.int32),
                       pltpu.VMEM((SLAB_E, H), _F32),
                       pltpu.VMEM((SLAB_E, H), _F32),
                       pltpu.SemaphoreType.DMA,
                       pltpu.SemaphoreType.DMA,
                       pltpu.SemaphoreType.DMA,
                       pltpu.SemaphoreType.DMA],
    )
    def gk(table_hbm, d_hbm, s_hbm, od_hbm, os_hbm,
           idx_d, idx_s, rows_d, rows_s, sem_d, sem_s, sem_od, sem_os):
        wid = lax.axis_index("s") * 2 + lax.axis_index("c")
        nslab = n_edges // SLAB_E
        pltpu.sync_copy(d_hbm.at[pl.ds(wid * SLAB_E, SLAB_E)], idx_d)
        pltpu.sync_copy(s_hbm.at[pl.ds(wid * SLAB_E, SLAB_E)], idx_s)

        @pl.loop(wid, nslab, step=NW)
        def _(k):
            base = k * SLAB_E

            @pl.when(k != wid)
            def _():
                pltpu.make_async_copy(
                    rows_d, od_hbm.at[pl.ds(base - NW * SLAB_E, SLAB_E)],
                    sem_od).wait()
                pltpu.make_async_copy(
                    rows_s, os_hbm.at[pl.ds(base - NW * SLAB_E, SLAB_E)],
                    sem_os).wait()

            gd = [pltpu.async_copy(table_hbm.at[idx_d.at[pl.ds(j * 128, 128)]],
                                   rows_d.at[pl.ds(j * 128, 128)], sem_d)
                  for j in range(GSLAB)]
            gs = [pltpu.async_copy(table_hbm.at[idx_s.at[pl.ds(j * 128, 128)]],
                                   rows_s.at[pl.ds(j * 128, 128)], sem_s)
                  for j in range(GSLAB)]
            for cp in gd:
                cp.wait()
            for cp in gs:
                cp.wait()
            pltpu.async_copy(rows_d, od_hbm.at[pl.ds(base, SLAB_E)], sem_od)
            pltpu.async_copy(rows_s, os_hbm.at[pl.ds(base, SLAB_E)], sem_os)

            @pl.when(k + NW < nslab)
            def _():
                pltpu.sync_copy(d_hbm.at[pl.ds(base + NW * SLAB_E, SLAB_E)],
                                idx_d)
                pltpu.sync_copy(s_hbm.at[pl.ds(base + NW * SLAB_E, SLAB_E)],
                                idx_s)

        pltpu.make_async_copy(rows_d, od_hbm.at[pl.ds(0, SLAB_E)],
                              sem_od).wait()
        pltpu.make_async_copy(rows_s, os_hbm.at[pl.ds(0, SLAB_E)],
                              sem_os).wait()

    return gk(table, dst1, src1)


def _sc_scatter_add(msg, dst2d, zeros, n_edges):
    nslab = n_edges // 128

    @functools.partial(
        pl.kernel,
        mesh=_sc_mesh(),
        out_type=jax.ShapeDtypeStruct((2 * NPAD, H), _F32),
        scratch_types=[pltpu.VMEM((1, 128), jnp.int32),
                       pltpu.VMEM((1, 128), jnp.int32),
                       pltpu.VMEM((128, H), _F32),
                       pltpu.VMEM((128, H), _F32),
                       pltpu.VMEM_SHARED((NPAD, H), _F32),
                       pltpu.SemaphoreType.DMA,
                       pltpu.SemaphoreType.DMA],
    )
    def sk(msg_hbm, d_hbm, z_hbm, out_hbm, idx_a, idx_b, rows_a, rows_b,
           acc, sem_a, sem_b):
        c = lax.axis_index("c")
        s = lax.axis_index("s")
        wid = s * 2 + c
        rows_per = NPAD // 16
        pltpu.sync_copy(z_hbm.at[pl.ds(s * rows_per, rows_per)],
                        acc.at[pl.ds(s * rows_per, rows_per)])
        plsc.subcore_barrier()

        def load(k, idx_v, rows_v, sem):
            pltpu.async_copy(d_hbm.at[pl.ds(k, 1)], idx_v, sem)
            pltpu.async_copy(msg_hbm.at[pl.ds(k * 128, 128)], rows_v, sem)

        def drain(k, idx_v, rows_v, sem):
            pltpu.make_async_copy(d_hbm.at[pl.ds(k, 1)], idx_v, sem).wait()
            pltpu.make_async_copy(msg_hbm.at[pl.ds(k * 128, 128)], rows_v,
                                  sem).wait()

        load(wid, idx_a, rows_a, sem_a)

        @pl.loop(wid, nslab, step=2 * NW)
        def _(k):
            kb = k + NW

            @pl.when(kb < nslab)
            def _():
                load(kb, idx_b, rows_b, sem_b)

            drain(k, idx_a, rows_a, sem_a)
            pltpu.sync_copy(rows_a, acc.at[idx_a.at[0]], add=True)

            @pl.when(k + 2 * NW < nslab)
            def _():
                load(k + 2 * NW, idx_a, rows_a, sem_a)

            @pl.when(kb < nslab)
            def _():
                drain(kb, idx_b, rows_b, sem_b)
                pltpu.sync_copy(rows_b, acc.at[idx_b.at[0]], add=True)

        plsc.subcore_barrier()
        pltpu.sync_copy(acc.at[pl.ds(s * rows_per, rows_per)],
                        out_hbm.at[pl.ds(c * NPAD + s * rows_per, rows_per)])

    return sk(msg, dst2d, zeros)


def kernel(V, E, edge_index, params):
    bounds = [sum(E_CHUNKS[:c]) for c in range(len(E_CHUNKS) + 1)]
    src1 = [edge_index[0][bounds[c]:bounds[c + 1]]
            for c in range(len(E_CHUNKS))]
    dst1 = [edge_index[1][bounds[c]:bounds[c + 1]]
            for c in range(len(E_CHUNKS))]
    dst2d = [d.reshape(-1, 128) for d in dst1]
    Vp = jnp.pad(V, ((0, NPAD - N_NODES), (0, 0)))
    zeros = jnp.zeros((NPAD, H), _F32)

    _sd = jax.ShapeDtypeStruct
    bf16 = jnp.bfloat16
    lV, lVb = _mlp_call(_enc_dual_body, [(Vp, 0)], _prep_mlp(params["enc_V"]),
                        [_sd((NPAD, H), _F32), _sd((NPAD, H), bf16)])
    lE = _mlp_call(_enc_bf_body, [(E, 0)], _prep_mlp(params["enc_E"]),
                   [_sd((N_EDGES, H), bf16)])

    nch = len(E_CHUNKS)
    lE = [lE] + [None] * (nch - 1)
    le_off = [sum(E_CHUNKS[:c]) // ROW_TILE for c in range(nch)]
    first = True
    for p in params["procs"]:
        wm = _prep_mlp(p["msg"])
        aggs, new_lE = [], [None] * nch
        for c in range(nch):
            ec = E_CHUNKS[c]
            Vd, Vs = _sc_gather_pair(lV, dst1[c], src1[c], ec)
            le_arr = lE[0] if first else lE[c]
            off = le_off[c] if first else 0
            msg, new_lE[c] = _mlp_call(
                _msg_f32_body, [(Vd, 0), (Vs, 0), (le_arr, off)], wm,
                [_sd((ec, H), _F32), _sd((ec, H), bf16)])
            aggs.append(_sc_scatter_add(msg, dst2d[c], zeros, ec))
        lE = new_lE
        first = False
        npb = NPAD // ROW_TILE
        parts = []
        for a in aggs:
            parts += [(a, 0), (a, npb)]
        lV, lVb = _mlp_call(_make_upd_body(2 * nch), parts + [(lV, 0)],
                            _prep_mlp(p["upd"]),
                            [_sd((NPAD, H), _F32), _sd((NPAD, H), bf16)])

    dec = _prep_mlp(params["dec"])
    dec[6] = jnp.pad(dec[6], ((0, 0), (0, 5)))
    dec[7] = jnp.pad(dec[7], ((0, 0), (0, 5)))
    out = _mlp_call(_dec_body, [(lV, 0)], dec, [_sd((NPAD, 8), _F32)])
    return out[:N_NODES, :3]


# speedup vs baseline: 2.6775x; 1.0018x over previous
import functools

import jax
import jax.numpy as jnp
from jax import lax
from jax.experimental import pallas as pl
from jax.experimental.pallas import tpu as pltpu
from jax.experimental.pallas import tpu_sc as plsc

N_NODES = 10000
NPAD = 10240
N_EDGES = 320000
H = 128
ROW_TILE = 512
GSLAB = 2
NW = 32
E_CHUNKS = (80384, 79872, 79872, 79872)

_F32 = jnp.float32


def _dot(x, w):
    return jnp.dot(x.astype(_F32), w, preferred_element_type=_F32)


def _leaky(x):
    return jnp.where(x >= 0, x, 0.01 * x)


def _layernorm(x, g, b):
    mu = jnp.mean(x, axis=-1, keepdims=True)
    xc = x - mu
    var = jnp.mean(xc * xc, axis=-1, keepdims=True)
    return xc * lax.rsqrt(var + 1e-5) * g + b


def _tail3(x, w1, b1, w2, b2, w3, b3, g, bln):
    x = _leaky(x)
    x = _dot(x, w1[...]) + b1[...]
    x = _leaky(x)
    x = _dot(x, w2[...]) + b2[...]
    x = _leaky(x)
    x = _dot(x, w3[...]) + b3[...]
    return _layernorm(x, g[...], bln[...])


def _enc_body(x_ref, w0, b0, w1, b1, w2, b2, w3, b3, g, bln, o_ref):
    x = _dot(x_ref[...], w0[...]) + b0[...]
    o_ref[...] = _tail3(x, w1, b1, w2, b2, w3, b3, g, bln)


def _enc_bf_body(x_ref, w0, b0, w1, b1, w2, b2, w3, b3, g, bln, o_ref):
    x = _dot(x_ref[...], w0[...]) + b0[...]
    o_ref[...] = _tail3(x, w1, b1, w2, b2, w3, b3, g, bln).astype(jnp.bfloat16)


def _msg_f32_body(vd, vs, le, w0, b0, w1, b1, w2, b2, w3, b3, g, bln,
                  msg_o, le_o):
    x = (_dot(vd[...], w0[0:H, :]) + _dot(vs[...], w0[H:2 * H, :])
         + _dot(le[...], w0[2 * H:3 * H, :]) + b0[...])
    m = _tail3(x, w1, b1, w2, b2, w3, b3, g, bln)
    msg_o[...] = m
    le_o[...] = (le[...].astype(_F32) + m).astype(jnp.bfloat16)


def _make_upd_body(n_agg):
    def body(*refs):
        aggs = refs[:n_agg]
        v = refs[n_agg]
        w0, b0, w1, b1, w2, b2, w3, b3, g, bln = refs[n_agg + 1:n_agg + 11]
        o_ref = refs[n_agg + 11]
        agg = aggs[0][...]
        for a in aggs[1:]:
            agg = agg + a[...]
        x = _dot(agg, w0[0:H, :]) + _dot(v[...], w0[H:2 * H, :]) + b0[...]
        o_ref[...] = v[...] + _tail3(x, w1, b1, w2, b2, w3, b3, g, bln)
    return body


def _dec_body(x_ref, w0, b0, w1, b1, w2, b2, w3, b3, o_ref):
    x = _leaky(_dot(x_ref[...], w0[...]) + b0[...])
    x = _leaky(_dot(x, w1[...]) + b1[...])
    x = _leaky(_dot(x, w2[...]) + b2[...])
    o_ref[...] = _dot(x, w3[...]) + b3[...]


def _mlp_call(body, parts, weights, outs):
    n_rows = outs[0].shape[0]
    grid = (n_rows // ROW_TILE,)

    def _row_spec(ncols, off):
        return pl.BlockSpec((ROW_TILE, ncols), lambda i, o=off: (i + o, 0))

    in_specs = ([_row_spec(p.shape[1], off) for p, off in parts]
                + [pl.BlockSpec(w.shape, lambda i: (0, 0)) for w in weights])
    out_specs = [_row_spec(o.shape[1], 0) for o in outs]
    single = len(outs) == 1
    res = pl.pallas_call(
        body,
        grid=grid,
        in_specs=in_specs,
        out_specs=out_specs[0] if single else out_specs,
        out_shape=outs[0] if single else outs,
        compiler_params=pltpu.CompilerParams(
            dimension_semantics=("parallel",)),
    )(*[p for p, _ in parts], *weights)
    return res


def _prep_mlp(p):
    ws = []
    for l in p["linears"]:
        ws.append(l["W"].T.astype(_F32))
        ws.append(l["b"].reshape(1, -1).astype(_F32))
    if "ln" in p:
        ws.append(p["ln"]["g"].reshape(1, -1).astype(_F32))
        ws.append(p["ln"]["b"].reshape(1, -1).astype(_F32))
    return ws


def _sc_mesh():
    return plsc.VectorSubcoreMesh(core_axis_name="c", subcore_axis_name="s")


def _sc_gather_pair(table, dst1, src1, n_edges):
    SLAB_E = GSLAB * 128

    @functools.partial(
        pl.kernel,
        mesh=_sc_mesh(),
        out_type=[jax.ShapeDtypeStruct((n_edges, H), _F32),
                  jax.ShapeDtypeStruct((n_edges, H), _F32)],
        scratch_types=[pltpu.VMEM((SLAB_E,), jnp.int32),
                       pltpu.VMEM((SLAB_E,), jnp.int32),
                       pltpu.VMEM((SLAB_E, H), _F32),
                       pltpu.VMEM((SLAB_E, H), _F32),
                       pltpu.SemaphoreType.DMA,
                       pltpu.SemaphoreType.DMA,
                       pltpu.SemaphoreType.DMA,
                       pltpu.SemaphoreType.DMA],
    )
    def gk(table_hbm, d_hbm, s_hbm, od_hbm, os_hbm,
           idx_d, idx_s, rows_d, rows_s, sem_d, sem_s, sem_od, sem_os):
        wid = lax.axis_index("s") * 2 + lax.axis_index("c")
        nslab = n_edges // SLAB_E
        pltpu.sync_copy(d_hbm.at[pl.ds(wid * SLAB_E, SLAB_E)], idx_d)
        pltpu.sync_copy(s_hbm.at[pl.ds(wid * SLAB_E, SLAB_E)], idx_s)

        @pl.loop(wid, nslab, step=NW)
        def _(k):
            base = k * SLAB_E

            @pl.when(k != wid)
            def _():
                pltpu.make_async_copy(
                    rows_d, od_hbm.at[pl.ds(base - NW * SLAB_E, SLAB_E)],
                    sem_od).wait()
                pltpu.make_async_copy(
                    rows_s, os_hbm.at[pl.ds(base - NW * SLAB_E, SLAB_E)],
                    sem_os).wait()

            gd = [pltpu.async_copy(table_hbm.at[idx_d.at[pl.ds(j * 128, 128)]],
                                   rows_d.at[pl.ds(j * 128, 128)], sem_d)
                  for j in range(GSLAB)]
            gs = [pltpu.async_copy(table_hbm.at[idx_s.at[pl.ds(j * 128, 128)]],
                                   rows_s.at[pl.ds(j * 128, 128)], sem_s)
                  for j in range(GSLAB)]
            for cp in gd:
                cp.wait()
            for cp in gs:
                cp.wait()
            pltpu.async_copy(rows_d, od_hbm.at[pl.ds(base, SLAB_E)], sem_od)
            pltpu.async_copy(rows_s, os_hbm.at[pl.ds(base, SLAB_E)], sem_os)

            @pl.when(k + NW < nslab)
            def _():
                pltpu.sync_copy(d_hbm.at[pl.ds(base + NW * SLAB_E, SLAB_E)],
                                idx_d)
                pltpu.sync_copy(s_hbm.at[pl.ds(base + NW * SLAB_E, SLAB_E)],
                                idx_s)

        pltpu.make_async_copy(rows_d, od_hbm.at[pl.ds(0, SLAB_E)],
                              sem_od).wait()
        pltpu.make_async_copy(rows_s, os_hbm.at[pl.ds(0, SLAB_E)],
                              sem_os).wait()

    return gk(table, dst1, src1)


def _sc_scatter_add(msg, dst2d, zeros, n_edges):
    nslab = n_edges // 128

    @functools.partial(
        pl.kernel,
        mesh=_sc_mesh(),
        out_type=jax.ShapeDtypeStruct((2 * NPAD, H), _F32),
        scratch_types=[pltpu.VMEM((1, 128), jnp.int32),
                       pltpu.VMEM((1, 128), jnp.int32),
                       pltpu.VMEM((128, H), _F32),
                       pltpu.VMEM((128, H), _F32),
                       pltpu.VMEM_SHARED((NPAD, H), _F32),
                       pltpu.SemaphoreType.DMA,
                       pltpu.SemaphoreType.DMA],
    )
    def sk(msg_hbm, d_hbm, z_hbm, out_hbm, idx_a, idx_b, rows_a, rows_b,
           acc, sem_a, sem_b):
        c = lax.axis_index("c")
        s = lax.axis_index("s")
        wid = s * 2 + c
        rows_per = NPAD // 16
        pltpu.sync_copy(z_hbm.at[pl.ds(s * rows_per, rows_per)],
                        acc.at[pl.ds(s * rows_per, rows_per)])
        plsc.subcore_barrier()

        def load(k, idx_v, rows_v, sem):
            pltpu.async_copy(d_hbm.at[pl.ds(k, 1)], idx_v, sem)
            pltpu.async_copy(msg_hbm.at[pl.ds(k * 128, 128)], rows_v, sem)

        def drain(k, idx_v, rows_v, sem):
            pltpu.make_async_copy(d_hbm.at[pl.ds(k, 1)], idx_v, sem).wait()
            pltpu.make_async_copy(msg_hbm.at[pl.ds(k * 128, 128)], rows_v,
                                  sem).wait()

        load(wid, idx_a, rows_a, sem_a)

        @pl.loop(wid, nslab, step=2 * NW)
        def _(k):
            kb = k + NW

            @pl.when(kb < nslab)
            def _():
                load(kb, idx_b, rows_b, sem_b)

            drain(k, idx_a, rows_a, sem_a)
            pltpu.sync_copy(rows_a, acc.at[idx_a.at[0]], add=True)

            @pl.when(k + 2 * NW < nslab)
            def _():
                load(k + 2 * NW, idx_a, rows_a, sem_a)

            @pl.when(kb < nslab)
            def _():
                drain(kb, idx_b, rows_b, sem_b)
                pltpu.sync_copy(rows_b, acc.at[idx_b.at[0]], add=True)

        plsc.subcore_barrier()
        pltpu.sync_copy(acc.at[pl.ds(s * rows_per, rows_per)],
                        out_hbm.at[pl.ds(c * NPAD + s * rows_per, rows_per)])

    return sk(msg, dst2d, zeros)


def kernel(V, E, edge_index, params):
    bounds = [sum(E_CHUNKS[:c]) for c in range(len(E_CHUNKS) + 1)]
    src1 = [edge_index[0][bounds[c]:bounds[c + 1]]
            for c in range(len(E_CHUNKS))]
    dst1 = [edge_index[1][bounds[c]:bounds[c + 1]]
            for c in range(len(E_CHUNKS))]
    dst2d = [d.reshape(-1, 128) for d in dst1]
    Vp = jnp.pad(V, ((0, NPAD - N_NODES), (0, 0)))
    zeros = jnp.zeros((NPAD, H), _F32)

    _sd = jax.ShapeDtypeStruct
    bf16 = jnp.bfloat16
    lV = _mlp_call(_enc_body, [(Vp, 0)], _prep_mlp(params["enc_V"]),
                   [_sd((NPAD, H), _F32)])
    lE = _mlp_call(_enc_bf_body, [(E, 0)], _prep_mlp(params["enc_E"]),
                   [_sd((N_EDGES, H), bf16)])

    nch = len(E_CHUNKS)
    lE = [lE] + [None] * (nch - 1)
    le_off = [sum(E_CHUNKS[:c]) // ROW_TILE for c in range(nch)]
    first = True
    for p in params["procs"]:
        wm = _prep_mlp(p["msg"])
        aggs, new_lE = [], [None] * nch
        for c in range(nch):
            ec = E_CHUNKS[c]
            Vd, Vs = _sc_gather_pair(lV, dst1[c], src1[c], ec)
            le_arr = lE[0] if first else lE[c]
            off = le_off[c] if first else 0
            msg, new_lE[c] = _mlp_call(
                _msg_f32_body, [(Vd, 0), (Vs, 0), (le_arr, off)], wm,
                [_sd((ec, H), _F32), _sd((ec, H), bf16)])
            aggs.append(_sc_scatter_add(msg, dst2d[c], zeros, ec))
        lE = new_lE
        first = False
        npb = NPAD // ROW_TILE
        parts = []
        for a in aggs:
            parts += [(a, 0), (a, npb)]
        lV = _mlp_call(_make_upd_body(2 * nch), parts + [(lV, 0)],
                       _prep_mlp(p["upd"]),
                       [_sd((NPAD, H), _F32)])

    dec = _prep_mlp(params["dec"])
    dec[6] = jnp.pad(dec[6], ((0, 0), (0, 5)))
    dec[7] = jnp.pad(dec[7], ((0, 0), (0, 5)))
    out = _mlp_call(_dec_body, [(lV, 0)], dec, [_sd((NPAD, 8), _F32)])
    return out[:N_NODES, :3]
